# Initial kernel scaffold; baseline (speedup 1.0000x reference)
import jax
import jax.numpy as jnp
from jax.experimental import pallas as pl


def kernel(x, edge_index, batch, W1, b1, W2, b2, W3, b3, Wl, bl):
    raise NotImplementedError("write your pallas kernel here")


# speedup vs baseline: 17.0162x; 17.0162x over previous
import functools

import jax
import jax.numpy as jnp
from jax import lax
from jax.experimental import pallas as pl
from jax.experimental.pallas import tpu as pltpu
from jax.experimental.pallas import tpu_sc as plsc

N_NODES = 10000
N_EDGES = 320000
N_GRAPHS = 64

NW = 32
B_EDGE = 128
N_IT = 79
E_PER_W = N_IT * B_EDGE
N_EDGES_PAD = NW * E_PER_W
N_EDGE_PAD = N_EDGES_PAD - N_EDGES
N_SUB = 16
N_PAD = 10240
ROWS_PER_SUB = N_PAD // N_SUB
DEG_D = 128

R_BLK = 2000
NB = N_NODES // R_BLK


def _mesh():
    return plsc.VectorSubcoreMesh(core_axis_name="c", subcore_axis_name="s")


@functools.lru_cache(maxsize=None)
def _make_agg():
    @functools.partial(
        pl.kernel,
        out_type=jax.ShapeDtypeStruct((2, N_PAD, 128), jnp.float32),
        mesh=_mesh(),
        scratch_types=[
            pltpu.VMEM((N_IT, B_EDGE), jnp.int32),
            pltpu.VMEM((N_IT, B_EDGE), jnp.int32),
            pltpu.VMEM((B_EDGE, 128), jnp.float32),
            pltpu.VMEM_SHARED((N_PAD, 128), jnp.float32),
            pltpu.SemaphoreType.DMA,
        ],
    )
    def agg(z_hbm, src_hbm, dst_hbm, zeros_hbm, out_hbm, src_v, dst_v,
            rows_v, acc, sem):
        cid = lax.axis_index("c")
        sid = lax.axis_index("s")
        wid = sid * 2 + cid
        pltpu.sync_copy(zeros_hbm, acc.at[pl.ds(sid * ROWS_PER_SUB, ROWS_PER_SUB)])
        pltpu.sync_copy(src_hbm.at[wid], src_v)
        pltpu.sync_copy(dst_hbm.at[wid], dst_v)
        plsc.subcore_barrier()

        def body(i, carry):
            pltpu.async_copy(z_hbm.at[src_v.at[i]], rows_v, sem).wait()
            pltpu.sync_copy(rows_v, acc.at[dst_v.at[i]], add=True)
            return carry

        lax.fori_loop(0, N_IT, body, 0)
        plsc.subcore_barrier()
        pltpu.sync_copy(acc.at[pl.ds(sid * ROWS_PER_SUB, ROWS_PER_SUB)],
                        out_hbm.at[cid, pl.ds(sid * ROWS_PER_SUB, ROWS_PER_SUB)])

    return agg


@functools.lru_cache(maxsize=None)
def _make_deg():
    @functools.partial(
        pl.kernel,
        out_type=jax.ShapeDtypeStruct((2, N_PAD, DEG_D), jnp.float32),
        mesh=_mesh(),
        scratch_types=[
            pltpu.VMEM((N_IT, B_EDGE), jnp.int32),
            pltpu.VMEM((B_EDGE, DEG_D), jnp.float32),
            pltpu.VMEM_SHARED((N_PAD, DEG_D), jnp.float32),
        ],
    )
    def degk(dst_hbm, ones_hbm, zeros_hbm, out_hbm, dst_v, ones_v, acc):
        cid = lax.axis_index("c")
        sid = lax.axis_index("s")
        wid = sid * 2 + cid
        pltpu.sync_copy(zeros_hbm, acc.at[pl.ds(sid * ROWS_PER_SUB, ROWS_PER_SUB)])
        pltpu.sync_copy(ones_hbm, ones_v)
        pltpu.sync_copy(dst_hbm.at[wid], dst_v)
        plsc.subcore_barrier()

        def body(i, carry):
            pltpu.sync_copy(ones_v, acc.at[dst_v.at[i]], add=True)
            return carry

        lax.fori_loop(0, N_IT, body, 0)
        plsc.subcore_barrier()
        pltpu.sync_copy(acc.at[pl.ds(sid * ROWS_PER_SUB, ROWS_PER_SUB)],
                        out_hbm.at[cid, pl.ds(sid * ROWS_PER_SUB, ROWS_PER_SUB)])

    return degk


def _dinv(d0_ref, d1_ref):
    deg = d0_ref[:, 0:1] + d1_ref[:, 0:1] + 1.0
    return lax.rsqrt(deg)


def _scale_matmul_body(x_ref, w_ref, d0_ref, d1_ref, o_ref):
    dinv = _dinv(d0_ref, d1_ref)
    o_ref[...] = jnp.dot(x_ref[...], w_ref[...],
                         preferred_element_type=jnp.float32) * dinv


def _combine_body(u0_ref, u1_ref, z_ref, d0_ref, d1_ref, b_ref, w_ref, o_ref):
    dinv = _dinv(d0_ref, d1_ref)
    h = (u0_ref[...] + u1_ref[...] + z_ref[...]) * dinv + b_ref[...]
    h = jnp.maximum(h, 0.0)
    o_ref[...] = jnp.dot(h, w_ref[...],
                         preferred_element_type=jnp.float32) * dinv


def _final_body(u0_ref, u1_ref, z_ref, d0_ref, d1_ref, b_ref, bt_ref,
                wl_ref, bl_ref, o_ref, s_acc, c_acc):
    i = pl.program_id(0)

    @pl.when(i == 0)
    def _():
        s_acc[...] = jnp.zeros_like(s_acc)
        c_acc[...] = jnp.zeros_like(c_acc)

    dinv = _dinv(d0_ref, d1_ref)
    h = (u0_ref[...] + u1_ref[...] + z_ref[...]) * dinv + b_ref[...]
    bt = bt_ref[0, 0, :]
    gids = lax.broadcasted_iota(jnp.int32, (N_GRAPHS, R_BLK), 0)
    mask = jnp.where(bt[None, :] == gids, 1.0, 0.0)
    s_acc[...] += jnp.dot(mask, h, preferred_element_type=jnp.float32)
    c_acc[...] += jnp.sum(mask, axis=1, keepdims=True)

    @pl.when(i == NB - 1)
    def _():
        g = s_acc[...] / jnp.maximum(c_acc[:, 0:1], 1.0)
        o_ref[...] = jnp.dot(g, wl_ref[...],
                             preferred_element_type=jnp.float32) + bl_ref[...]


def _scale_matmul(x, w, d0, d1):
    din, dout = w.shape
    return pl.pallas_call(
        _scale_matmul_body,
        grid=(NB,),
        in_specs=[
            pl.BlockSpec((R_BLK, din), lambda i: (i, 0)),
            pl.BlockSpec((din, dout), lambda i: (0, 0)),
            pl.BlockSpec((R_BLK, 8), lambda i: (i, 0)),
            pl.BlockSpec((R_BLK, 8), lambda i: (i, 0)),
        ],
        out_specs=pl.BlockSpec((R_BLK, dout), lambda i: (i, 0)),
        out_shape=jax.ShapeDtypeStruct((N_NODES, dout), jnp.float32),
    )(x, w, d0, d1)


def _combine(u0, u1, z, d0, d1, b, w):
    din, dout = w.shape
    return pl.pallas_call(
        _combine_body,
        grid=(NB,),
        in_specs=[
            pl.BlockSpec((R_BLK, din), lambda i: (i, 0)),
            pl.BlockSpec((R_BLK, din), lambda i: (i, 0)),
            pl.BlockSpec((R_BLK, din), lambda i: (i, 0)),
            pl.BlockSpec((R_BLK, 8), lambda i: (i, 0)),
            pl.BlockSpec((R_BLK, 8), lambda i: (i, 0)),
            pl.BlockSpec((1, din), lambda i: (0, 0)),
            pl.BlockSpec((din, dout), lambda i: (0, 0)),
        ],
        out_specs=pl.BlockSpec((R_BLK, dout), lambda i: (i, 0)),
        out_shape=jax.ShapeDtypeStruct((N_NODES, dout), jnp.float32),
    )(u0, u1, z, d0, d1, b, w)


def _final(u0, u1, z, d0, d1, b, bt, wl, bl):
    din = z.shape[1]
    return pl.pallas_call(
        _final_body,
        grid=(NB,),
        in_specs=[
            pl.BlockSpec((R_BLK, din), lambda i: (i, 0)),
            pl.BlockSpec((R_BLK, din), lambda i: (i, 0)),
            pl.BlockSpec((R_BLK, din), lambda i: (i, 0)),
            pl.BlockSpec((R_BLK, 8), lambda i: (i, 0)),
            pl.BlockSpec((R_BLK, 8), lambda i: (i, 0)),
            pl.BlockSpec((1, din), lambda i: (0, 0)),
            pl.BlockSpec((1, 1, R_BLK), lambda i: (i, 0, 0)),
            pl.BlockSpec((din, 2), lambda i: (0, 0)),
            pl.BlockSpec((1, 2), lambda i: (0, 0)),
        ],
        out_specs=pl.BlockSpec((N_GRAPHS, 2), lambda i: (0, 0)),
        out_shape=jax.ShapeDtypeStruct((N_GRAPHS, 2), jnp.float32),
        scratch_shapes=[
            pltpu.VMEM((N_GRAPHS, 128), jnp.float32),
            pltpu.VMEM((N_GRAPHS, 128), jnp.float32),
        ],
    )(u0, u1, z, d0, d1, b, bt, wl, bl)


def _deg_partials(dst3):
    ones = jnp.ones((B_EDGE, DEG_D), jnp.float32)
    zeros = jnp.zeros((ROWS_PER_SUB, DEG_D), jnp.float32)
    return _make_deg()(dst3, ones, zeros)


def _pad_mat(w, rows, cols):
    return jnp.zeros((rows, cols), jnp.float32).at[:w.shape[0], :w.shape[1]].set(w)


def _pad_vec(b, n):
    return jnp.zeros((1, n), jnp.float32).at[0, :b.shape[0]].set(b)


def _agg_call(z, src3, dst3):
    zeros = jnp.zeros((ROWS_PER_SUB, 128), jnp.float32)
    return _make_agg()(z, src3, dst3, zeros)


def kernel(x, edge_index, batch, W1, b1, W2, b2, W3, b3, Wl, bl):
    e = jnp.arange(N_EDGE_PAD, dtype=jnp.int32)
    pad_src = (e * 7919) % N_NODES
    pad_dst = N_NODES + (e % (N_PAD - N_NODES))
    src3 = jnp.concatenate([edge_index[0].astype(jnp.int32), pad_src]
                           ).reshape(NW, N_IT, B_EDGE)
    dst3 = jnp.concatenate([edge_index[1].astype(jnp.int32), pad_dst]
                           ).reshape(NW, N_IT, B_EDGE)

    degp = _deg_partials(dst3)
    d0, d1 = degp[0, :, :8], degp[1, :, :8]

    W2p = _pad_mat(W2, 128, 128)
    W3p = _pad_mat(W3, 128, 128)
    Wlp = _pad_mat(Wl, 128, 2)

    z1 = _scale_matmul(x, W1, d0, d1)
    u1 = _agg_call(z1, src3, dst3)
    z2 = _combine(u1[0], u1[1], z1, d0, d1, b1.reshape(1, -1), W2p)
    u2 = _agg_call(z2, src3, dst3)
    z3 = _combine(u2[0], u2[1], z2, d0, d1, _pad_vec(b2, 128), W3p)
    u3 = _agg_call(z3, src3, dst3)
    out = _final(u3[0], u3[1], z3, d0, d1, _pad_vec(b3, 128),
                 batch.astype(jnp.int32).reshape(NB, 1, R_BLK),
                 Wlp, bl.reshape(1, 2))
    return out


# speedup vs baseline: 20.7330x; 1.2184x over previous
import functools

import jax
import jax.numpy as jnp
from jax import lax
from jax.experimental import pallas as pl
from jax.experimental.pallas import tpu as pltpu
from jax.experimental.pallas import tpu_sc as plsc

N_NODES = 10000
N_EDGES = 320000
N_GRAPHS = 64

NW = 32
B_EDGE = 128
N_IT = 80
E_PER_W = N_IT * B_EDGE
N_EDGES_PAD = NW * E_PER_W
N_EDGE_PAD = N_EDGES_PAD - N_EDGES
N_SUB = 16
N_PAD = 10240
ROWS_PER_SUB = N_PAD // N_SUB
DEG_D = 128

R_BLK = 2000
NB = N_NODES // R_BLK


def _mesh():
    return plsc.VectorSubcoreMesh(core_axis_name="c", subcore_axis_name="s")


@functools.lru_cache(maxsize=None)
def _make_agg():
    @functools.partial(
        pl.kernel,
        out_type=jax.ShapeDtypeStruct((2, N_PAD, 128), jnp.float32),
        mesh=_mesh(),
        scratch_types=[
            pltpu.VMEM((N_IT // 2, B_EDGE), jnp.int32),
            pltpu.VMEM((N_IT // 2, B_EDGE), jnp.int32),
            pltpu.VMEM((B_EDGE, 128), jnp.float32),
            pltpu.VMEM((B_EDGE, 128), jnp.float32),
            pltpu.VMEM_SHARED((N_PAD, 128), jnp.float32),
            pltpu.SemaphoreType.DMA,
        ],
    )
    def agg(z_hbm, src_hbm, dst_hbm, zeros_hbm, out_hbm, src_v, dst_v,
            rows0, rows1, acc, sem0):
        cid = lax.axis_index("c")
        sid = lax.axis_index("s")
        wid = sid * 2 + cid
        half = N_IT // 2
        pltpu.sync_copy(zeros_hbm, acc.at[pl.ds(sid * ROWS_PER_SUB, ROWS_PER_SUB)])
        plsc.subcore_barrier()

        for ph in range(2):
            pltpu.sync_copy(src_hbm.at[wid, pl.ds(ph * half, half)], src_v)
            pltpu.sync_copy(dst_hbm.at[wid, pl.ds(ph * half, half)], dst_v)
            pltpu.async_copy(z_hbm.at[src_v.at[0]], rows0, sem0)

            def body(j, carry):
                i0 = 2 * j
                i1 = i0 + 1
                i2 = jnp.minimum(i0 + 2, half - 1)
                pltpu.make_async_copy(z_hbm.at[src_v.at[i0]], rows0, sem0).wait()
                pltpu.async_copy(z_hbm.at[src_v.at[i1]], rows1, sem0)
                pltpu.sync_copy(rows0, acc.at[dst_v.at[i0]], add=True)
                pltpu.make_async_copy(z_hbm.at[src_v.at[i1]], rows1, sem0).wait()
                pltpu.async_copy(z_hbm.at[src_v.at[i2]], rows0, sem0)
                pltpu.sync_copy(rows1, acc.at[dst_v.at[i1]], add=True)
                return carry

            lax.fori_loop(0, half // 2, body, 0)
            pltpu.make_async_copy(z_hbm.at[src_v.at[half - 1]], rows0, sem0).wait()

        plsc.subcore_barrier()
        pltpu.sync_copy(acc.at[pl.ds(sid * ROWS_PER_SUB, ROWS_PER_SUB)],
                        out_hbm.at[cid, pl.ds(sid * ROWS_PER_SUB, ROWS_PER_SUB)])

    return agg


@functools.lru_cache(maxsize=None)
def _make_deg():
    @functools.partial(
        pl.kernel,
        out_type=jax.ShapeDtypeStruct((2, N_PAD, DEG_D), jnp.float32),
        mesh=_mesh(),
        scratch_types=[
            pltpu.VMEM((N_IT, B_EDGE), jnp.int32),
            pltpu.VMEM((B_EDGE, DEG_D), jnp.float32),
            pltpu.VMEM_SHARED((N_PAD, DEG_D), jnp.float32),
        ],
    )
    def degk(dst_hbm, ones_hbm, zeros_hbm, out_hbm, dst_v, ones_v, acc):
        cid = lax.axis_index("c")
        sid = lax.axis_index("s")
        wid = sid * 2 + cid
        pltpu.sync_copy(zeros_hbm, acc.at[pl.ds(sid * ROWS_PER_SUB, ROWS_PER_SUB)])
        pltpu.sync_copy(ones_hbm, ones_v)
        pltpu.sync_copy(dst_hbm.at[wid], dst_v)
        plsc.subcore_barrier()

        def body(i, carry):
            pltpu.sync_copy(ones_v, acc.at[dst_v.at[i]], add=True)
            return carry

        lax.fori_loop(0, N_IT, body, 0)
        plsc.subcore_barrier()
        pltpu.sync_copy(acc.at[pl.ds(sid * ROWS_PER_SUB, ROWS_PER_SUB)],
                        out_hbm.at[cid, pl.ds(sid * ROWS_PER_SUB, ROWS_PER_SUB)])

    return degk


def _dinv(d0_ref, d1_ref):
    deg = d0_ref[:, 0:1] + d1_ref[:, 0:1] + 1.0
    return lax.rsqrt(deg)


def _scale_matmul_body(x_ref, w_ref, d0_ref, d1_ref, o_ref):
    dinv = _dinv(d0_ref, d1_ref)
    o_ref[...] = jnp.dot(x_ref[...], w_ref[...],
                         preferred_element_type=jnp.float32) * dinv


def _combine_body(u0_ref, u1_ref, z_ref, d0_ref, d1_ref, b_ref, w_ref, o_ref):
    dinv = _dinv(d0_ref, d1_ref)
    h = (u0_ref[...] + u1_ref[...] + z_ref[...]) * dinv + b_ref[...]
    h = jnp.maximum(h, 0.0)
    o_ref[...] = jnp.dot(h, w_ref[...],
                         preferred_element_type=jnp.float32) * dinv


def _final_body(u0_ref, u1_ref, z_ref, d0_ref, d1_ref, b_ref, bt_ref,
                wl_ref, bl_ref, o_ref, s_acc, c_acc):
    i = pl.program_id(0)

    @pl.when(i == 0)
    def _():
        s_acc[...] = jnp.zeros_like(s_acc)
        c_acc[...] = jnp.zeros_like(c_acc)

    dinv = _dinv(d0_ref, d1_ref)
    h = (u0_ref[...] + u1_ref[...] + z_ref[...]) * dinv + b_ref[...]
    bt = bt_ref[0, 0, :]
    gids = lax.broadcasted_iota(jnp.int32, (N_GRAPHS, R_BLK), 0)
    mask = jnp.where(bt[None, :] == gids, 1.0, 0.0)
    s_acc[...] += jnp.dot(mask, h, preferred_element_type=jnp.float32)
    c_acc[...] += jnp.sum(mask, axis=1, keepdims=True)

    @pl.when(i == NB - 1)
    def _():
        g = s_acc[...] / jnp.maximum(c_acc[:, 0:1], 1.0)
        o_ref[...] = jnp.dot(g, wl_ref[...],
                             preferred_element_type=jnp.float32) + bl_ref[...]


def _scale_matmul(x, w, d0, d1):
    din, dout = w.shape
    return pl.pallas_call(
        _scale_matmul_body,
        grid=(NB,),
        in_specs=[
            pl.BlockSpec((R_BLK, din), lambda i: (i, 0)),
            pl.BlockSpec((din, dout), lambda i: (0, 0)),
            pl.BlockSpec((R_BLK, 8), lambda i: (i, 0)),
            pl.BlockSpec((R_BLK, 8), lambda i: (i, 0)),
        ],
        out_specs=pl.BlockSpec((R_BLK, dout), lambda i: (i, 0)),
        out_shape=jax.ShapeDtypeStruct((N_NODES, dout), jnp.float32),
    )(x, w, d0, d1)


def _combine(u0, u1, z, d0, d1, b, w):
    din, dout = w.shape
    return pl.pallas_call(
        _combine_body,
        grid=(NB,),
        in_specs=[
            pl.BlockSpec((R_BLK, din), lambda i: (i, 0)),
            pl.BlockSpec((R_BLK, din), lambda i: (i, 0)),
            pl.BlockSpec((R_BLK, din), lambda i: (i, 0)),
            pl.BlockSpec((R_BLK, 8), lambda i: (i, 0)),
            pl.BlockSpec((R_BLK, 8), lambda i: (i, 0)),
            pl.BlockSpec((1, din), lambda i: (0, 0)),
            pl.BlockSpec((din, dout), lambda i: (0, 0)),
        ],
        out_specs=pl.BlockSpec((R_BLK, dout), lambda i: (i, 0)),
        out_shape=jax.ShapeDtypeStruct((N_NODES, dout), jnp.float32),
    )(u0, u1, z, d0, d1, b, w)


def _final(u0, u1, z, d0, d1, b, bt, wl, bl):
    din = z.shape[1]
    return pl.pallas_call(
        _final_body,
        grid=(NB,),
        in_specs=[
            pl.BlockSpec((R_BLK, din), lambda i: (i, 0)),
            pl.BlockSpec((R_BLK, din), lambda i: (i, 0)),
            pl.BlockSpec((R_BLK, din), lambda i: (i, 0)),
            pl.BlockSpec((R_BLK, 8), lambda i: (i, 0)),
            pl.BlockSpec((R_BLK, 8), lambda i: (i, 0)),
            pl.BlockSpec((1, din), lambda i: (0, 0)),
            pl.BlockSpec((1, 1, R_BLK), lambda i: (i, 0, 0)),
            pl.BlockSpec((din, 2), lambda i: (0, 0)),
            pl.BlockSpec((1, 2), lambda i: (0, 0)),
        ],
        out_specs=pl.BlockSpec((N_GRAPHS, 2), lambda i: (0, 0)),
        out_shape=jax.ShapeDtypeStruct((N_GRAPHS, 2), jnp.float32),
        scratch_shapes=[
            pltpu.VMEM((N_GRAPHS, 128), jnp.float32),
            pltpu.VMEM((N_GRAPHS, 128), jnp.float32),
        ],
    )(u0, u1, z, d0, d1, b, bt, wl, bl)


def _deg_partials(dst3):
    ones = jnp.ones((B_EDGE, DEG_D), jnp.float32)
    zeros = jnp.zeros((ROWS_PER_SUB, DEG_D), jnp.float32)
    return _make_deg()(dst3, ones, zeros)


def _pad_mat(w, rows, cols):
    return jnp.zeros((rows, cols), jnp.float32).at[:w.shape[0], :w.shape[1]].set(w)


def _pad_vec(b, n):
    return jnp.zeros((1, n), jnp.float32).at[0, :b.shape[0]].set(b)


def _agg_call(z, src3, dst3):
    zeros = jnp.zeros((ROWS_PER_SUB, 128), jnp.float32)
    return _make_agg()(z, src3, dst3, zeros)


def kernel(x, edge_index, batch, W1, b1, W2, b2, W3, b3, Wl, bl):
    e = jnp.arange(N_EDGE_PAD, dtype=jnp.int32)
    pad_src = (e * 7919) % N_NODES
    pad_dst = N_NODES + (e % (N_PAD - N_NODES))
    src3 = jnp.concatenate([edge_index[0].astype(jnp.int32), pad_src]
                           ).reshape(NW, N_IT, B_EDGE)
    dst3 = jnp.concatenate([edge_index[1].astype(jnp.int32), pad_dst]
                           ).reshape(NW, N_IT, B_EDGE)

    degp = _deg_partials(dst3)
    d0, d1 = degp[0, :, :8], degp[1, :, :8]

    W2p = _pad_mat(W2, 128, 128)
    W3p = _pad_mat(W3, 128, 128)
    Wlp = _pad_mat(Wl, 128, 2)

    z1 = _scale_matmul(x, W1, d0, d1)
    u1 = _agg_call(z1, src3, dst3)
    z2 = _combine(u1[0], u1[1], z1, d0, d1, b1.reshape(1, -1), W2p)
    u2 = _agg_call(z2, src3, dst3)
    z3 = _combine(u2[0], u2[1], z2, d0, d1, _pad_vec(b2, 128), W3p)
    u3 = _agg_call(z3, src3, dst3)
    out = _final(u3[0], u3[1], z3, d0, d1, _pad_vec(b3, 128),
                 batch.astype(jnp.int32).reshape(NB, 1, R_BLK),
                 Wlp, bl.reshape(1, 2))
    return out


# speedup vs baseline: 26.1056x; 1.2591x over previous
import functools

import jax
import jax.numpy as jnp
from jax import lax
from jax.experimental import pallas as pl
from jax.experimental.pallas import tpu as pltpu
from jax.experimental.pallas import tpu_sc as plsc

N_NODES = 10000
N_EDGES = 320000
N_GRAPHS = 64

NW = 32
B_EDGE = 128
N_IT = 80
E_PER_W = N_IT * B_EDGE
N_EDGES_PAD = NW * E_PER_W
N_EDGE_PAD = N_EDGES_PAD - N_EDGES
N_SUB = 16
N_PAD = 10240
ROWS_PER_SUB = N_PAD // N_SUB
DEG_D = 16

R_BLK = 2000
NB = N_NODES // R_BLK


def _mesh():
    return plsc.VectorSubcoreMesh(core_axis_name="c", subcore_axis_name="s")


@functools.lru_cache(maxsize=None)
def _make_agg(D):
    @functools.partial(
        pl.kernel,
        out_type=jax.ShapeDtypeStruct((2, N_PAD, D), jnp.float32),
        mesh=_mesh(),
        compiler_params=pltpu.CompilerParams(use_tc_tiling_on_sc=False),
        scratch_types=[
            pltpu.VMEM((N_IT // 2, B_EDGE), jnp.int32),
            pltpu.VMEM((N_IT // 2, B_EDGE), jnp.int32),
            pltpu.VMEM((B_EDGE, D), jnp.float32),
            pltpu.VMEM((B_EDGE, D), jnp.float32),
            pltpu.VMEM_SHARED((N_PAD, D), jnp.float32),
            pltpu.SemaphoreType.DMA,
        ],
    )
    def agg(z_hbm, src_hbm, dst_hbm, zeros_hbm, out_hbm, src_v, dst_v,
            rows0, rows1, acc, sem0):
        cid = lax.axis_index("c")
        sid = lax.axis_index("s")
        wid = sid * 2 + cid
        half = N_IT // 2
        pltpu.sync_copy(zeros_hbm, acc.at[pl.ds(sid * ROWS_PER_SUB, ROWS_PER_SUB)])
        plsc.subcore_barrier()

        for ph in range(2):
            pltpu.sync_copy(src_hbm.at[wid, pl.ds(ph * half, half)], src_v)
            pltpu.sync_copy(dst_hbm.at[wid, pl.ds(ph * half, half)], dst_v)
            pltpu.async_copy(z_hbm.at[src_v.at[0]], rows0, sem0)

            def body(j, carry):
                i0 = 2 * j
                i1 = i0 + 1
                i2 = jnp.minimum(i0 + 2, half - 1)
                pltpu.make_async_copy(z_hbm.at[src_v.at[i0]], rows0, sem0).wait()
                pltpu.async_copy(z_hbm.at[src_v.at[i1]], rows1, sem0)
                pltpu.sync_copy(rows0, acc.at[dst_v.at[i0]], add=True)
                pltpu.make_async_copy(z_hbm.at[src_v.at[i1]], rows1, sem0).wait()
                pltpu.async_copy(z_hbm.at[src_v.at[i2]], rows0, sem0)
                pltpu.sync_copy(rows1, acc.at[dst_v.at[i1]], add=True)
                return carry

            lax.fori_loop(0, half // 2, body, 0)
            pltpu.make_async_copy(z_hbm.at[src_v.at[half - 1]], rows0, sem0).wait()

        plsc.subcore_barrier()
        pltpu.sync_copy(acc.at[pl.ds(sid * ROWS_PER_SUB, ROWS_PER_SUB)],
                        out_hbm.at[cid, pl.ds(sid * ROWS_PER_SUB, ROWS_PER_SUB)])

    return agg


@functools.lru_cache(maxsize=None)
def _make_deg():
    @functools.partial(
        pl.kernel,
        out_type=jax.ShapeDtypeStruct((2, N_PAD, DEG_D), jnp.float32),
        mesh=_mesh(),
        compiler_params=pltpu.CompilerParams(use_tc_tiling_on_sc=False),
        scratch_types=[
            pltpu.VMEM((N_IT, B_EDGE), jnp.int32),
            pltpu.VMEM((B_EDGE, DEG_D), jnp.float32),
            pltpu.VMEM_SHARED((N_PAD, DEG_D), jnp.float32),
        ],
    )
    def degk(dst_hbm, ones_hbm, zeros_hbm, out_hbm, dst_v, ones_v, acc):
        cid = lax.axis_index("c")
        sid = lax.axis_index("s")
        wid = sid * 2 + cid
        pltpu.sync_copy(zeros_hbm, acc.at[pl.ds(sid * ROWS_PER_SUB, ROWS_PER_SUB)])
        pltpu.sync_copy(ones_hbm, ones_v)
        pltpu.sync_copy(dst_hbm.at[wid], dst_v)
        plsc.subcore_barrier()

        def body(i, carry):
            pltpu.sync_copy(ones_v, acc.at[dst_v.at[i]], add=True)
            return carry

        lax.fori_loop(0, N_IT, body, 0)
        plsc.subcore_barrier()
        pltpu.sync_copy(acc.at[pl.ds(sid * ROWS_PER_SUB, ROWS_PER_SUB)],
                        out_hbm.at[cid, pl.ds(sid * ROWS_PER_SUB, ROWS_PER_SUB)])

    return degk


def _dinv(d0_ref, d1_ref):
    deg = d0_ref[:, 0:1] + d1_ref[:, 0:1] + 1.0
    return lax.rsqrt(deg)


def _scale_matmul_body(x_ref, w_ref, d0_ref, d1_ref, o_ref):
    dinv = _dinv(d0_ref, d1_ref)
    o_ref[...] = jnp.dot(x_ref[...], w_ref[...],
                         preferred_element_type=jnp.float32) * dinv


def _combine_body(u0_ref, u1_ref, z_ref, d0_ref, d1_ref, b_ref, w_ref, o_ref):
    dinv = _dinv(d0_ref, d1_ref)
    h = (u0_ref[...] + u1_ref[...] + z_ref[...]) * dinv + b_ref[...]
    h = jnp.maximum(h, 0.0)
    o_ref[...] = jnp.dot(h, w_ref[...],
                         preferred_element_type=jnp.float32) * dinv


def _final_body(u0_ref, u1_ref, z_ref, d0_ref, d1_ref, b_ref, bt_ref,
                wl_ref, bl_ref, o_ref, s_acc, c_acc):
    i = pl.program_id(0)

    @pl.when(i == 0)
    def _():
        s_acc[...] = jnp.zeros_like(s_acc)
        c_acc[...] = jnp.zeros_like(c_acc)

    dinv = _dinv(d0_ref, d1_ref)
    h = (u0_ref[...] + u1_ref[...] + z_ref[...]) * dinv + b_ref[...]
    bt = bt_ref[0, 0, :]
    gids = lax.broadcasted_iota(jnp.int32, (N_GRAPHS, R_BLK), 0)
    mask = jnp.where(bt[None, :] == gids, 1.0, 0.0)
    s_acc[...] += jnp.dot(mask, h, preferred_element_type=jnp.float32)
    c_acc[...] += jnp.sum(mask, axis=1, keepdims=True)

    @pl.when(i == NB - 1)
    def _():
        g = s_acc[...] / jnp.maximum(c_acc[:, 0:1], 1.0)
        o_ref[...] = jnp.dot(g, wl_ref[...],
                             preferred_element_type=jnp.float32) + bl_ref[...]


def _scale_matmul(x, w, d0, d1):
    din, dout = w.shape
    return pl.pallas_call(
        _scale_matmul_body,
        grid=(NB,),
        in_specs=[
            pl.BlockSpec((R_BLK, din), lambda i: (i, 0)),
            pl.BlockSpec((din, dout), lambda i: (0, 0)),
            pl.BlockSpec((R_BLK, 8), lambda i: (i, 0)),
            pl.BlockSpec((R_BLK, 8), lambda i: (i, 0)),
        ],
        out_specs=pl.BlockSpec((R_BLK, dout), lambda i: (i, 0)),
        out_shape=jax.ShapeDtypeStruct((N_NODES, dout), jnp.float32),
    )(x, w, d0, d1)


def _combine(u0, u1, z, d0, d1, b, w):
    din, dout = w.shape
    return pl.pallas_call(
        _combine_body,
        grid=(NB,),
        in_specs=[
            pl.BlockSpec((R_BLK, din), lambda i: (i, 0)),
            pl.BlockSpec((R_BLK, din), lambda i: (i, 0)),
            pl.BlockSpec((R_BLK, din), lambda i: (i, 0)),
            pl.BlockSpec((R_BLK, 8), lambda i: (i, 0)),
            pl.BlockSpec((R_BLK, 8), lambda i: (i, 0)),
            pl.BlockSpec((1, din), lambda i: (0, 0)),
            pl.BlockSpec((din, dout), lambda i: (0, 0)),
        ],
        out_specs=pl.BlockSpec((R_BLK, dout), lambda i: (i, 0)),
        out_shape=jax.ShapeDtypeStruct((N_NODES, dout), jnp.float32),
    )(u0, u1, z, d0, d1, b, w)


def _final(u0, u1, z, d0, d1, b, bt, wl, bl):
    din = z.shape[1]
    return pl.pallas_call(
        _final_body,
        grid=(NB,),
        in_specs=[
            pl.BlockSpec((R_BLK, din), lambda i: (i, 0)),
            pl.BlockSpec((R_BLK, din), lambda i: (i, 0)),
            pl.BlockSpec((R_BLK, din), lambda i: (i, 0)),
            pl.BlockSpec((R_BLK, 8), lambda i: (i, 0)),
            pl.BlockSpec((R_BLK, 8), lambda i: (i, 0)),
            pl.BlockSpec((1, din), lambda i: (0, 0)),
            pl.BlockSpec((1, 1, R_BLK), lambda i: (i, 0, 0)),
            pl.BlockSpec((din, 2), lambda i: (0, 0)),
            pl.BlockSpec((1, 2), lambda i: (0, 0)),
        ],
        out_specs=pl.BlockSpec((N_GRAPHS, 2), lambda i: (0, 0)),
        out_shape=jax.ShapeDtypeStruct((N_GRAPHS, 2), jnp.float32),
        scratch_shapes=[
            pltpu.VMEM((N_GRAPHS, 32), jnp.float32),
            pltpu.VMEM((N_GRAPHS, 128), jnp.float32),
        ],
    )(u0, u1, z, d0, d1, b, bt, wl, bl)


def _deg_partials(dst3):
    ones = jnp.ones((B_EDGE, DEG_D), jnp.float32)
    zeros = jnp.zeros((ROWS_PER_SUB, DEG_D), jnp.float32)
    return _make_deg()(dst3, ones, zeros)


def _pad_mat(w, rows, cols):
    return jnp.zeros((rows, cols), jnp.float32).at[:w.shape[0], :w.shape[1]].set(w)


def _pad_vec(b, n):
    return jnp.zeros((1, n), jnp.float32).at[0, :b.shape[0]].set(b)


def _agg_call(z, src3, dst3):
    D = z.shape[1]
    zeros = jnp.zeros((ROWS_PER_SUB, D), jnp.float32)
    return _make_agg(D)(z, src3, dst3, zeros)


def kernel(x, edge_index, batch, W1, b1, W2, b2, W3, b3, Wl, bl):
    e = jnp.arange(N_EDGE_PAD, dtype=jnp.int32)
    pad_src = (e * 7919) % N_NODES
    pad_dst = N_NODES + (e % (N_PAD - N_NODES))
    src3 = jnp.concatenate([edge_index[0].astype(jnp.int32), pad_src]
                           ).reshape(NW, N_IT, B_EDGE)
    dst3 = jnp.concatenate([edge_index[1].astype(jnp.int32), pad_dst]
                           ).reshape(NW, N_IT, B_EDGE)

    degp = _deg_partials(dst3)
    d0, d1 = degp[0, :, :8], degp[1, :, :8]

    z1 = _scale_matmul(x, W1, d0, d1)
    u1 = _agg_call(z1, src3, dst3)
    z2 = _combine(u1[0], u1[1], z1, d0, d1, b1.reshape(1, -1), W2)
    u2 = _agg_call(z2, src3, dst3)
    z3 = _combine(u2[0], u2[1], z2, d0, d1, b2.reshape(1, -1), W3)
    u3 = _agg_call(z3, src3, dst3)
    out = _final(u3[0], u3[1], z3, d0, d1, b3.reshape(1, -1),
                 batch.astype(jnp.int32).reshape(NB, 1, R_BLK),
                 Wl, bl.reshape(1, 2))
    return out


# speedup vs baseline: 31.0220x; 1.1883x over previous
import functools

import jax
import jax.numpy as jnp
from jax import lax
from jax.experimental import pallas as pl
from jax.experimental.pallas import tpu as pltpu
from jax.experimental.pallas import tpu_sc as plsc

N_NODES = 10000
N_EDGES = 320000
N_GRAPHS = 64

NW = 32
B_EDGE = 128
N_IT = 80
E_PER_W = N_IT * B_EDGE
N_EDGES_PAD = NW * E_PER_W
N_EDGE_PAD = N_EDGES_PAD - N_EDGES
N_SUB = 16
N_PAD = 10240
ROWS_PER_SUB = N_PAD // N_SUB
DEG_D = 16

R_BLK = 2000
NB = N_NODES // R_BLK


def _mesh():
    return plsc.VectorSubcoreMesh(core_axis_name="c", subcore_axis_name="s")


@functools.lru_cache(maxsize=None)
def _make_agg(D):
    be = 256 if D <= 64 else 128
    n_it = E_PER_W // be
    params = (pltpu.CompilerParams(use_tc_tiling_on_sc=False)
              if D <= 64 else pltpu.CompilerParams())

    @functools.partial(
        pl.kernel,
        out_type=jax.ShapeDtypeStruct((2, N_PAD, D), jnp.float32),
        mesh=_mesh(),
        compiler_params=params,
        scratch_types=[
            pltpu.VMEM((n_it // 2, be), jnp.int32),
            pltpu.VMEM((n_it // 2, be), jnp.int32),
            pltpu.VMEM((be, D), jnp.float32),
            pltpu.VMEM((be, D), jnp.float32),
            pltpu.VMEM_SHARED((N_PAD, D), jnp.float32),
            pltpu.SemaphoreType.DMA,
        ],
    )
    def agg(z_hbm, src_hbm, dst_hbm, zeros_hbm, out_hbm, src_v, dst_v,
            rows0, rows1, acc, sem0):
        cid = lax.axis_index("c")
        sid = lax.axis_index("s")
        wid = sid * 2 + cid
        half = n_it // 2
        pltpu.sync_copy(zeros_hbm, acc.at[pl.ds(sid * ROWS_PER_SUB, ROWS_PER_SUB)])
        plsc.subcore_barrier()

        for ph in range(2):
            pltpu.sync_copy(src_hbm.at[wid, pl.ds(ph * half, half)], src_v)
            pltpu.sync_copy(dst_hbm.at[wid, pl.ds(ph * half, half)], dst_v)
            pltpu.async_copy(z_hbm.at[src_v.at[0]], rows0, sem0)

            def body(j, carry):
                i0 = 2 * j
                i1 = i0 + 1
                i2 = jnp.minimum(i0 + 2, half - 1)
                pltpu.make_async_copy(z_hbm.at[src_v.at[i0]], rows0, sem0).wait()
                pltpu.async_copy(z_hbm.at[src_v.at[i1]], rows1, sem0)
                pltpu.sync_copy(rows0, acc.at[dst_v.at[i0]], add=True)
                pltpu.make_async_copy(z_hbm.at[src_v.at[i1]], rows1, sem0).wait()
                pltpu.async_copy(z_hbm.at[src_v.at[i2]], rows0, sem0)
                pltpu.sync_copy(rows1, acc.at[dst_v.at[i1]], add=True)
                return carry

            lax.fori_loop(0, half // 2, body, 0)
            pltpu.make_async_copy(z_hbm.at[src_v.at[half - 1]], rows0, sem0).wait()

        plsc.subcore_barrier()
        pltpu.sync_copy(acc.at[pl.ds(sid * ROWS_PER_SUB, ROWS_PER_SUB)],
                        out_hbm.at[cid, pl.ds(sid * ROWS_PER_SUB, ROWS_PER_SUB)])

    return agg


@functools.lru_cache(maxsize=None)
def _make_deg():
    @functools.partial(
        pl.kernel,
        out_type=jax.ShapeDtypeStruct((2, N_PAD, DEG_D), jnp.float32),
        mesh=_mesh(),
        compiler_params=pltpu.CompilerParams(use_tc_tiling_on_sc=False),
        scratch_types=[
            pltpu.VMEM((E_PER_W // 256, 256), jnp.int32),
            pltpu.VMEM((256, DEG_D), jnp.float32),
            pltpu.VMEM_SHARED((N_PAD, DEG_D), jnp.float32),
        ],
    )
    def degk(dst_hbm, ones_hbm, zeros_hbm, out_hbm, dst_v, ones_v, acc):
        cid = lax.axis_index("c")
        sid = lax.axis_index("s")
        wid = sid * 2 + cid
        pltpu.sync_copy(zeros_hbm, acc.at[pl.ds(sid * ROWS_PER_SUB, ROWS_PER_SUB)])
        pltpu.sync_copy(ones_hbm, ones_v)
        pltpu.sync_copy(dst_hbm.at[wid], dst_v)
        plsc.subcore_barrier()

        def body(i, carry):
            pltpu.sync_copy(ones_v, acc.at[dst_v.at[i]], add=True)
            return carry

        lax.fori_loop(0, E_PER_W // 256, body, 0)
        plsc.subcore_barrier()
        pltpu.sync_copy(acc.at[pl.ds(sid * ROWS_PER_SUB, ROWS_PER_SUB)],
                        out_hbm.at[cid, pl.ds(sid * ROWS_PER_SUB, ROWS_PER_SUB)])

    return degk


def _dinv(dp_ref):
    deg = dp_ref[0, :, 0:1] + dp_ref[1, :, 0:1] + 1.0
    return lax.rsqrt(deg)


def _scale_matmul_body(x_ref, w_ref, dp_ref, o_ref):
    dinv = _dinv(dp_ref)
    o_ref[...] = jnp.dot(x_ref[...], w_ref[...],
                         preferred_element_type=jnp.float32) * dinv


def _combine_body(u0_ref, u1_ref, z_ref, dp_ref, b_ref, w_ref, o_ref):
    dinv = _dinv(dp_ref)
    h = (u0_ref[0] + u1_ref[0] + z_ref[...]) * dinv + b_ref[...]
    h = jnp.maximum(h, 0.0)
    o_ref[...] = jnp.dot(h, w_ref[...],
                         preferred_element_type=jnp.float32) * dinv


def _final_body(u0_ref, u1_ref, z_ref, dp_ref, b_ref, bt_ref,
                wl_ref, bl_ref, o_ref, s_acc, c_acc):
    i = pl.program_id(0)

    @pl.when(i == 0)
    def _():
        s_acc[...] = jnp.zeros_like(s_acc)
        c_acc[...] = jnp.zeros_like(c_acc)

    dinv = _dinv(dp_ref)
    h = (u0_ref[0] + u1_ref[0] + z_ref[...]) * dinv + b_ref[...]
    bt = bt_ref[0, 0, :]
    gids = lax.broadcasted_iota(jnp.int32, (N_GRAPHS, R_BLK), 0)
    mask = jnp.where(bt[None, :] == gids, 1.0, 0.0)
    s_acc[...] += jnp.dot(mask, h, preferred_element_type=jnp.float32)
    c_acc[...] += jnp.sum(mask, axis=1, keepdims=True)

    @pl.when(i == NB - 1)
    def _():
        g = s_acc[...] / jnp.maximum(c_acc[:, 0:1], 1.0)
        o_ref[...] = jnp.dot(g, wl_ref[...],
                             preferred_element_type=jnp.float32) + bl_ref[...]


def _scale_matmul(x, w, degp):
    din, dout = w.shape
    return pl.pallas_call(
        _scale_matmul_body,
        grid=(NB,),
        in_specs=[
            pl.BlockSpec((R_BLK, din), lambda i: (i, 0)),
            pl.BlockSpec((din, dout), lambda i: (0, 0)),
            pl.BlockSpec((2, R_BLK, DEG_D), lambda i: (0, i, 0)),
        ],
        out_specs=pl.BlockSpec((R_BLK, dout), lambda i: (i, 0)),
        out_shape=jax.ShapeDtypeStruct((N_NODES, dout), jnp.float32),
    )(x, w, degp)


def _combine(u, z, degp, b, w):
    din, dout = w.shape
    return pl.pallas_call(
        _combine_body,
        grid=(NB,),
        in_specs=[
            pl.BlockSpec((1, R_BLK, din), lambda i: (0, i, 0)),
            pl.BlockSpec((1, R_BLK, din), lambda i: (1, i, 0)),
            pl.BlockSpec((R_BLK, din), lambda i: (i, 0)),
            pl.BlockSpec((2, R_BLK, DEG_D), lambda i: (0, i, 0)),
            pl.BlockSpec((1, din), lambda i: (0, 0)),
            pl.BlockSpec((din, dout), lambda i: (0, 0)),
        ],
        out_specs=pl.BlockSpec((R_BLK, dout), lambda i: (i, 0)),
        out_shape=jax.ShapeDtypeStruct((N_NODES, dout), jnp.float32),
    )(u, u, z, degp, b, w)


def _final(u, z, degp, b, bt, wl, bl):
    din = z.shape[1]
    return pl.pallas_call(
        _final_body,
        grid=(NB,),
        in_specs=[
            pl.BlockSpec((1, R_BLK, din), lambda i: (0, i, 0)),
            pl.BlockSpec((1, R_BLK, din), lambda i: (1, i, 0)),
            pl.BlockSpec((R_BLK, din), lambda i: (i, 0)),
            pl.BlockSpec((2, R_BLK, DEG_D), lambda i: (0, i, 0)),
            pl.BlockSpec((1, din), lambda i: (0, 0)),
            pl.BlockSpec((1, 1, R_BLK), lambda i: (i, 0, 0)),
            pl.BlockSpec((din, 2), lambda i: (0, 0)),
            pl.BlockSpec((1, 2), lambda i: (0, 0)),
        ],
        out_specs=pl.BlockSpec((N_GRAPHS, 2), lambda i: (0, 0)),
        out_shape=jax.ShapeDtypeStruct((N_GRAPHS, 2), jnp.float32),
        scratch_shapes=[
            pltpu.VMEM((N_GRAPHS, 32), jnp.float32),
            pltpu.VMEM((N_GRAPHS, 128), jnp.float32),
        ],
    )(u, u, z, degp, b, bt, wl, bl)


def _deg_partials(dst3):
    ones = jnp.ones((256, DEG_D), jnp.float32)
    zeros = jnp.zeros((ROWS_PER_SUB, DEG_D), jnp.float32)
    return _make_deg()(dst3.reshape(NW, -1, 256), ones, zeros)


def _pad_mat(w, rows, cols):
    return jnp.zeros((rows, cols), jnp.float32).at[:w.shape[0], :w.shape[1]].set(w)


def _pad_vec(b, n):
    return jnp.zeros((1, n), jnp.float32).at[0, :b.shape[0]].set(b)


def _agg_call(z, src3, dst3):
    D = z.shape[1]
    be = 256 if D <= 64 else 128
    zeros = jnp.zeros((ROWS_PER_SUB, D), jnp.float32)
    return _make_agg(D)(z, src3.reshape(NW, -1, be), dst3.reshape(NW, -1, be),
                        zeros)


def kernel(x, edge_index, batch, W1, b1, W2, b2, W3, b3, Wl, bl):
    e = jnp.arange(N_EDGE_PAD, dtype=jnp.int32)
    pad_src = e & 8191
    pad_dst = N_NODES + (e & 127)
    src3 = jnp.concatenate([edge_index[0].astype(jnp.int32), pad_src]
                           ).reshape(NW, N_IT, B_EDGE)
    dst3 = jnp.concatenate([edge_index[1].astype(jnp.int32), pad_dst]
                           ).reshape(NW, N_IT, B_EDGE)

    degp = _deg_partials(dst3)

    z1 = _scale_matmul(x, W1, degp)
    u1 = _agg_call(z1, src3, dst3)
    z2 = _combine(u1, z1, degp, b1.reshape(1, -1), W2)
    u2 = _agg_call(z2, src3, dst3)
    z3 = _combine(u2, z2, degp, b2.reshape(1, -1), W3)
    u3 = _agg_call(z3, src3, dst3)
    out = _final(u3, z3, degp, b3.reshape(1, -1),
                 batch.astype(jnp.int32).reshape(NB, 1, R_BLK),
                 Wl, bl.reshape(1, 2))
    return out


# speedup vs baseline: 31.0391x; 1.0006x over previous
import functools

import jax
import jax.numpy as jnp
from jax import lax
from jax.experimental import pallas as pl
from jax.experimental.pallas import tpu as pltpu
from jax.experimental.pallas import tpu_sc as plsc

N_NODES = 10000
N_EDGES = 320000
N_GRAPHS = 64

NW = 32
B_EDGE = 128
N_IT = 80
E_PER_W = N_IT * B_EDGE
N_EDGES_PAD = NW * E_PER_W
N_EDGE_PAD = N_EDGES_PAD - N_EDGES
N_SUB = 16
N_PAD = 10240
ROWS_PER_SUB = N_PAD // N_SUB
DEG_D = 16

R_BLK = 2000
NB = N_NODES // R_BLK


def _mesh():
    return plsc.VectorSubcoreMesh(core_axis_name="c", subcore_axis_name="s")


@functools.lru_cache(maxsize=None)
def _make_agg(D):
    be = 256 if D <= 64 else 128
    n_it = E_PER_W // be
    params = (pltpu.CompilerParams(use_tc_tiling_on_sc=False)
              if D <= 64 else pltpu.CompilerParams())

    @functools.partial(
        pl.kernel,
        out_type=jax.ShapeDtypeStruct((2, N_PAD, D), jnp.float32),
        mesh=_mesh(),
        compiler_params=params,
        scratch_types=[
            pltpu.VMEM((n_it // 2, be), jnp.int32),
            pltpu.VMEM((n_it // 2, be), jnp.int32),
            pltpu.VMEM((be, D), jnp.float32),
            pltpu.VMEM((be, D), jnp.float32),
            pltpu.VMEM_SHARED((N_PAD, D), jnp.float32),
            pltpu.SemaphoreType.DMA,
        ],
    )
    def agg(z_hbm, src_hbm, dst_hbm, zeros_hbm, out_hbm, src_v, dst_v,
            rows0, rows1, acc, sem0):
        cid = lax.axis_index("c")
        sid = lax.axis_index("s")
        wid = sid * 2 + cid
        half = n_it // 2
        pltpu.sync_copy(zeros_hbm, acc.at[pl.ds(sid * ROWS_PER_SUB, ROWS_PER_SUB)])
        plsc.subcore_barrier()

        for ph in range(2):
            pltpu.sync_copy(src_hbm.at[wid, pl.ds(ph * half, half)], src_v)
            pltpu.sync_copy(dst_hbm.at[wid, pl.ds(ph * half, half)], dst_v)
            pltpu.async_copy(z_hbm.at[src_v.at[0]], rows0, sem0)

            def body(j, carry):
                i0 = 2 * j
                i1 = i0 + 1
                i2 = jnp.minimum(i0 + 2, half - 1)
                pltpu.make_async_copy(z_hbm.at[src_v.at[i0]], rows0, sem0).wait()
                pltpu.async_copy(z_hbm.at[src_v.at[i1]], rows1, sem0)
                pltpu.sync_copy(rows0, acc.at[dst_v.at[i0]], add=True)
                pltpu.make_async_copy(z_hbm.at[src_v.at[i1]], rows1, sem0).wait()
                pltpu.async_copy(z_hbm.at[src_v.at[i2]], rows0, sem0)
                pltpu.sync_copy(rows1, acc.at[dst_v.at[i1]], add=True)
                return carry

            lax.fori_loop(0, half // 2, body, 0)
            pltpu.make_async_copy(z_hbm.at[src_v.at[half - 1]], rows0, sem0).wait()

        plsc.subcore_barrier()
        pltpu.sync_copy(acc.at[pl.ds(sid * ROWS_PER_SUB, ROWS_PER_SUB)],
                        out_hbm.at[cid, pl.ds(sid * ROWS_PER_SUB, ROWS_PER_SUB)])

    return agg


@functools.lru_cache(maxsize=None)
def _make_deg():
    @functools.partial(
        pl.kernel,
        out_type=jax.ShapeDtypeStruct((2, N_PAD, DEG_D), jnp.float32),
        mesh=_mesh(),
        compiler_params=pltpu.CompilerParams(use_tc_tiling_on_sc=False),
        scratch_types=[
            pltpu.VMEM((E_PER_W // 256, 256), jnp.int32),
            pltpu.VMEM((256, DEG_D), jnp.float32),
            pltpu.VMEM_SHARED((N_PAD, DEG_D), jnp.float32),
        ],
    )
    def degk(dst_hbm, ones_hbm, zeros_hbm, out_hbm, dst_v, ones_v, acc):
        cid = lax.axis_index("c")
        sid = lax.axis_index("s")
        wid = sid * 2 + cid
        pltpu.sync_copy(zeros_hbm, acc.at[pl.ds(sid * ROWS_PER_SUB, ROWS_PER_SUB)])
        pltpu.sync_copy(ones_hbm, ones_v)
        pltpu.sync_copy(dst_hbm.at[wid], dst_v)
        plsc.subcore_barrier()

        def body(i, carry):
            pltpu.sync_copy(ones_v, acc.at[dst_v.at[i]], add=True)
            return carry

        lax.fori_loop(0, E_PER_W // 256, body, 0)
        plsc.subcore_barrier()
        pltpu.sync_copy(acc.at[pl.ds(sid * ROWS_PER_SUB, ROWS_PER_SUB)],
                        out_hbm.at[cid, pl.ds(sid * ROWS_PER_SUB, ROWS_PER_SUB)])

    return degk


def _dinv(dp_ref):
    deg = dp_ref[0, :, 0:1] + dp_ref[1, :, 0:1] + 1.0
    return lax.rsqrt(deg)


def _scale_matmul_body(x_ref, w_ref, dp_ref, o_ref):
    dinv = _dinv(dp_ref)
    o_ref[...] = jnp.dot(x_ref[...], w_ref[...],
                         preferred_element_type=jnp.float32) * dinv


def _combine_body(u0_ref, u1_ref, z_ref, dp_ref, b_ref, w_ref, o_ref):
    dinv = _dinv(dp_ref)
    h = (u0_ref[0] + u1_ref[0] + z_ref[...]) * dinv + b_ref[...]
    h = jnp.maximum(h, 0.0)
    o_ref[...] = jnp.dot(h, w_ref[...],
                         preferred_element_type=jnp.float32) * dinv


def _final_body(u0_ref, u1_ref, z_ref, dp_ref, b_ref, bt_ref,
                wl_ref, bl_ref, o_ref, s_acc, c_acc):
    i = pl.program_id(0)

    @pl.when(i == 0)
    def _():
        s_acc[...] = jnp.zeros_like(s_acc)
        c_acc[...] = jnp.zeros_like(c_acc)

    dinv = _dinv(dp_ref)
    h = (u0_ref[0] + u1_ref[0] + z_ref[...]) * dinv + b_ref[...]
    bt = bt_ref[0, 0, :]
    gids = lax.broadcasted_iota(jnp.int32, (N_GRAPHS, R_BLK), 0)
    mask = jnp.where(bt[None, :] == gids, 1.0, 0.0)
    s_acc[...] += jnp.dot(mask, h, preferred_element_type=jnp.float32)
    c_acc[...] += jnp.sum(mask, axis=1, keepdims=True)

    @pl.when(i == NB - 1)
    def _():
        g = s_acc[...] / jnp.maximum(c_acc[:, 0:1], 1.0)
        o_ref[...] = jnp.dot(g, wl_ref[...],
                             preferred_element_type=jnp.float32) + bl_ref[...]


def _scale_matmul(x, w, degp):
    din, dout = w.shape
    return pl.pallas_call(
        _scale_matmul_body,
        grid=(NB,),
        in_specs=[
            pl.BlockSpec((R_BLK, din), lambda i: (i, 0)),
            pl.BlockSpec((din, dout), lambda i: (0, 0)),
            pl.BlockSpec((2, R_BLK, DEG_D), lambda i: (0, i, 0)),
        ],
        out_specs=pl.BlockSpec((R_BLK, dout), lambda i: (i, 0)),
        out_shape=jax.ShapeDtypeStruct((N_NODES, dout), jnp.float32),
    )(x, w, degp)


def _combine(u, z, degp, b, w):
    din, dout = w.shape
    return pl.pallas_call(
        _combine_body,
        grid=(NB,),
        in_specs=[
            pl.BlockSpec((1, R_BLK, din), lambda i: (0, i, 0)),
            pl.BlockSpec((1, R_BLK, din), lambda i: (1, i, 0)),
            pl.BlockSpec((R_BLK, din), lambda i: (i, 0)),
            pl.BlockSpec((2, R_BLK, DEG_D), lambda i: (0, i, 0)),
            pl.BlockSpec((1, din), lambda i: (0, 0)),
            pl.BlockSpec((din, dout), lambda i: (0, 0)),
        ],
        out_specs=pl.BlockSpec((R_BLK, dout), lambda i: (i, 0)),
        out_shape=jax.ShapeDtypeStruct((N_NODES, dout), jnp.float32),
    )(u, u, z, degp, b, w)


def _final(u, z, degp, b, bt, wl, bl):
    din = z.shape[1]
    return pl.pallas_call(
        _final_body,
        grid=(NB,),
        in_specs=[
            pl.BlockSpec((1, R_BLK, din), lambda i: (0, i, 0)),
            pl.BlockSpec((1, R_BLK, din), lambda i: (1, i, 0)),
            pl.BlockSpec((R_BLK, din), lambda i: (i, 0)),
            pl.BlockSpec((2, R_BLK, DEG_D), lambda i: (0, i, 0)),
            pl.BlockSpec((1, din), lambda i: (0, 0)),
            pl.BlockSpec((1, 1, R_BLK), lambda i: (i, 0, 0)),
            pl.BlockSpec((din, 2), lambda i: (0, 0)),
            pl.BlockSpec((1, 2), lambda i: (0, 0)),
        ],
        out_specs=pl.BlockSpec((N_GRAPHS, 2), lambda i: (0, 0)),
        out_shape=jax.ShapeDtypeStruct((N_GRAPHS, 2), jnp.float32),
        scratch_shapes=[
            pltpu.VMEM((N_GRAPHS, 32), jnp.float32),
            pltpu.VMEM((N_GRAPHS, 128), jnp.float32),
        ],
    )(u, u, z, degp, b, bt, wl, bl)


def _deg_partials(dst3):
    ones = jnp.ones((256, DEG_D), jnp.float32)
    zeros = jnp.zeros((ROWS_PER_SUB, DEG_D), jnp.float32)
    return _make_deg()(dst3.reshape(NW, -1, 256), ones, zeros)


def _agg_call(z, src3, dst3):
    D = z.shape[1]
    be = 256 if D <= 64 else 128
    zeros = jnp.zeros((ROWS_PER_SUB, D), jnp.float32)
    return _make_agg(D)(z, src3.reshape(NW, -1, be), dst3.reshape(NW, -1, be),
                        zeros)


def kernel(x, edge_index, batch, W1, b1, W2, b2, W3, b3, Wl, bl):
    e = jnp.arange(N_EDGE_PAD, dtype=jnp.int32)
    pad_src = e & 8191
    pad_dst = N_NODES + (e & 127)
    src3 = jnp.concatenate([edge_index[0].astype(jnp.int32), pad_src]
                           ).reshape(NW, N_IT, B_EDGE)
    dst3 = jnp.concatenate([edge_index[1].astype(jnp.int32), pad_dst]
                           ).reshape(NW, N_IT, B_EDGE)

    degp = _deg_partials(dst3)

    z1 = _scale_matmul(x, W1, degp)
    u1 = _agg_call(z1, src3, dst3)
    z2 = _combine(u1, z1, degp, b1.reshape(1, -1), W2)
    u2 = _agg_call(z2, src3, dst3)
    z3 = _combine(u2, z2, degp, b2.reshape(1, -1), W3)
    u3 = _agg_call(z3, src3, dst3)
    out = _final(u3, z3, degp, b3.reshape(1, -1),
                 batch.astype(jnp.int32).reshape(NB, 1, R_BLK),
                 Wl, bl.reshape(1, 2))
    return out
